# baseline (speedup 1.0000x reference)
import jax
import jax.numpy as jnp
from jax import lax
from jax.experimental import pallas as pl
from jax.experimental.pallas import tpu as pltpu
from jax.experimental.pallas import tpu_sc as plsc

_N_TILES = 4
_SPLIT = 22
_K = 4

_NC, _NS = 2, 16
_NW = _NC * _NS
_ROWS = 32
_D = 1280
_CH = _D // 16
_LAST_ROW = 1024


def _sc_body(ar_ref, gate_ref, x_hbm, local_hbm, gpe_hbm, out_hbm,
             buf, lsc, gbuf, xrow, lrow, grow, arv, gv):
    cid = lax.axis_index("c")
    sid = lax.axis_index("s")
    wid = sid * _NC + cid
    base = wid * _ROWS
    is_last = wid == _NW - 1

    pltpu.sync_copy(gate_ref, gv)
    pltpu.sync_copy(ar_ref, arv)
    g16 = gv[...]
    e2g = jnp.exp(2.0 * g16)
    tgv = 1.0 - 2.0 / (e2g + 1.0)
    av = 1.0 - tgv
    gate_on = g16[0] != 0.0
    ar_v = arv[...]

    pltpu.sync_copy(local_hbm.at[pl.ds(base, _ROWS), :], lsc)

    def _scale_row(r, _):
        def _scale_chunk(c, _):
            sl = pl.ds(c * 16, 16)
            lsc[r, sl] = lsc[r, sl] * av
            return 0
        return lax.fori_loop(0, _CH, _scale_chunk, 0)

    lax.fori_loop(0, _ROWS, _scale_row, 0)

    @pl.when(is_last)
    def _scale_last():
        pltpu.sync_copy(local_hbm.at[pl.ds(_LAST_ROW, 1), :], lrow)

        def _scale_chunk(c, _):
            sl = pl.ds(c * 16, 16)
            lrow[0, sl] = lrow[0, sl] * av
            return 0
        lax.fori_loop(0, _CH, _scale_chunk, 0)

    for s in range(_SPLIT, 8 * _N_TILES):
        b = s // _N_TILES
        t = s % _N_TILES
        h = ar_v[2 * b]
        w = ar_v[2 * b + 1]
        w_safe = jnp.maximum(w, 1)
        row = t // w_safe
        col = t % w_safe
        valid = t < h * w
        fetch = jnp.logical_and(valid, gate_on)

        pltpu.sync_copy(x_hbm.at[b, t, pl.ds(base, _ROWS), :], buf)

        @pl.when(fetch)
        def _with_global():
            pltpu.sync_copy(gpe_hbm.at[row, col, pl.ds(base, _ROWS), :], gbuf)

            def _row(r, _):
                def _chunk(c, _):
                    sl = pl.ds(c * 16, 16)
                    buf[r, sl] = buf[r, sl] + lsc[r, sl] + gbuf[r, sl] * tgv
                    return 0
                return lax.fori_loop(0, _CH, _chunk, 0)
            lax.fori_loop(0, _ROWS, _row, 0)

        @pl.when(jnp.logical_not(fetch))
        def _local_only():
            def _row(r, _):
                def _chunk(c, _):
                    sl = pl.ds(c * 16, 16)
                    buf[r, sl] = buf[r, sl] + lsc[r, sl]
                    return 0
                return lax.fori_loop(0, _CH, _chunk, 0)
            lax.fori_loop(0, _ROWS, _row, 0)

        pltpu.sync_copy(buf, out_hbm.at[b, t, pl.ds(base, _ROWS), :])

        @pl.when(is_last)
        def _last_row():
            pltpu.sync_copy(x_hbm.at[b, t, pl.ds(_LAST_ROW, 1), :], xrow)

            @pl.when(fetch)
            def _wg():
                pltpu.sync_copy(gpe_hbm.at[row, col, pl.ds(_LAST_ROW, 1), :], grow)

                def _chunk(c, _):
                    sl = pl.ds(c * 16, 16)
                    xrow[0, sl] = xrow[0, sl] + lrow[0, sl] + grow[0, sl] * tgv
                    return 0
                lax.fori_loop(0, _CH, _chunk, 0)

            @pl.when(jnp.logical_not(fetch))
            def _lo():
                def _chunk(c, _):
                    sl = pl.ds(c * 16, 16)
                    xrow[0, sl] = xrow[0, sl] + lrow[0, sl]
                    return 0
                lax.fori_loop(0, _CH, _chunk, 0)

            pltpu.sync_copy(xrow, out_hbm.at[b, t, pl.ds(_LAST_ROW, 1), :])


def _sc_kernel(x, ar16, gpe, local, gate16):
    mesh = plsc.VectorSubcoreMesh(core_axis_name="c", subcore_axis_name="s")
    return pl.kernel(
        _sc_body,
        out_type=jax.ShapeDtypeStruct(x.shape, x.dtype),
        mesh=mesh,
        scratch_types=[
            pltpu.VMEM((_ROWS, _D), jnp.float32),
            pltpu.VMEM((_ROWS, _D), jnp.float32),
            pltpu.VMEM((_ROWS, _D), jnp.float32),
            pltpu.VMEM((1, _D), jnp.float32),
            pltpu.VMEM((1, _D), jnp.float32),
            pltpu.VMEM((1, _D), jnp.float32),
            pltpu.VMEM((16,), jnp.int32),
            pltpu.VMEM((16,), jnp.float32),
        ],
    )(ar16, gate16, x, local, gpe)


def _tc_body(ar_ref, gate_ref, x_hbm, local_hbm, gpe_hbm, prev_hbm, out_hbm,
             ibuf, obuf, lbuf, gchunk, isem, osem, lsem, gsem):
    tg = jnp.tanh(gate_ref[0])
    a = 1.0 - tg
    gate_on = tg != 0.0

    def in_copy(k, slot):
        b = k // _N_TILES
        t = k % _N_TILES
        return pltpu.make_async_copy(
            x_hbm.at[b, t], ibuf.at[slot], isem.at[slot])

    def out_copy(k, slot):
        b = k // _N_TILES
        t = k % _N_TILES
        return pltpu.make_async_copy(
            obuf.at[slot], out_hbm.at[b, t], osem.at[slot])

    pltpu.make_async_copy(local_hbm, lbuf, lsem).start()

    for k in range(_K - 1):
        in_copy(k, k).start()

    pltpu.make_async_copy(local_hbm, lbuf, lsem).wait()

    def step(k, carry):
        slot = k % _K
        b = k // _N_TILES
        t = k % _N_TILES
        h = ar_ref[b, 0]
        w = ar_ref[b, 1]
        w_safe = jnp.maximum(w, 1)
        row = t // w_safe
        col = t % w_safe
        valid = t < h * w
        fetch = jnp.logical_and(valid, gate_on)

        in_copy(k, slot).wait()

        @pl.when(k >= _K)
        def _free_out_slot():
            out_copy(k - _K, slot).wait()

        @pl.when(fetch)
        def _fetch_global():
            g = pltpu.make_async_copy(gpe_hbm.at[row, col], gchunk, gsem)
            g.start()
            g.wait()

        obuf[slot] = ibuf[slot] + lbuf[...] * a

        @pl.when(fetch)
        def _add_global():
            obuf[slot] += gchunk[...] * tg

        out_copy(k, slot).start()

        @pl.when(k + _K - 1 < _SPLIT)
        def _prefetch():
            in_copy(k + _K - 1, (k + _K - 1) % _K).start()

        return carry

    jax.lax.fori_loop(0, _SPLIT, step, 0)

    for k in range(_SPLIT - _K, _SPLIT):
        out_copy(k, k % _K).wait()


def _tc_kernel(x, ar, gpe, local, gate, prev):
    bsz, n_tiles, num_tokens, embed_dim = x.shape
    return pl.pallas_call(
        _tc_body,
        in_specs=[
            pl.BlockSpec(memory_space=pltpu.SMEM),
            pl.BlockSpec(memory_space=pltpu.SMEM),
            pl.BlockSpec(memory_space=pltpu.MemorySpace.HBM),
            pl.BlockSpec(memory_space=pltpu.MemorySpace.HBM),
            pl.BlockSpec(memory_space=pltpu.MemorySpace.HBM),
            pl.BlockSpec(memory_space=pltpu.MemorySpace.HBM),
        ],
        out_specs=pl.BlockSpec(memory_space=pltpu.MemorySpace.HBM),
        out_shape=jax.ShapeDtypeStruct((bsz, n_tiles, num_tokens, embed_dim), x.dtype),
        input_output_aliases={5: 0},
        scratch_shapes=[
            pltpu.VMEM((_K, num_tokens, embed_dim), jnp.float32),
            pltpu.VMEM((_K, num_tokens, embed_dim), jnp.float32),
            pltpu.VMEM((num_tokens, embed_dim), jnp.float32),
            pltpu.VMEM((num_tokens, embed_dim), jnp.float32),
            pltpu.SemaphoreType.DMA((_K,)),
            pltpu.SemaphoreType.DMA((_K,)),
            pltpu.SemaphoreType.DMA,
            pltpu.SemaphoreType.DMA,
        ],
    )(ar, gate, x, local, gpe, prev)


def kernel(x, aspect_ratio, global_positional_embedding, local_positional_embedding, gate):
    ar = aspect_ratio.astype(jnp.int32)
    ar16 = ar.reshape(16)
    gate16 = jnp.broadcast_to(gate.astype(jnp.float32), (16,))
    sc_out = _sc_kernel(x, ar16, global_positional_embedding,
                        local_positional_embedding, gate16)
    return _tc_kernel(x, ar, global_positional_embedding,
                      local_positional_embedding, gate, sc_out)


# speedup vs baseline: 1.3315x; 1.3315x over previous
import jax
import jax.numpy as jnp
from jax.experimental import pallas as pl
from jax.experimental.pallas import tpu as pltpu

_N_TILES = 4
_N_SLICES = 32
_K = 4


def _body(ar_ref, gate_ref, x_hbm, local_hbm, gpe_hbm, out_hbm,
          ibuf, obuf, lbuf, gchunk, isem, osem, lsem, gsem):
    tg = jnp.tanh(gate_ref[0])
    a = 1.0 - tg
    gate_on = tg != 0.0

    def in_copy(k, slot):
        b = k // _N_TILES
        t = k % _N_TILES
        return pltpu.make_async_copy(
            x_hbm.at[b, t], ibuf.at[slot], isem.at[slot])

    def out_copy(k, slot):
        b = k // _N_TILES
        t = k % _N_TILES
        return pltpu.make_async_copy(
            obuf.at[slot], out_hbm.at[b, t], osem.at[slot])

    pltpu.make_async_copy(local_hbm, lbuf, lsem).start()

    for k in range(_K - 1):
        in_copy(k, k).start()

    pltpu.make_async_copy(local_hbm, lbuf, lsem).wait()
    lbuf[...] = lbuf[...] * a

    def step(k, carry):
        slot = k % _K
        b = k // _N_TILES
        t = k % _N_TILES
        h = ar_ref[b, 0]
        w = ar_ref[b, 1]
        w_safe = jnp.maximum(w, 1)
        row = t // w_safe
        col = t % w_safe
        valid = t < h * w
        fetch = jnp.logical_and(valid, gate_on)

        in_copy(k, slot).wait()

        @pl.when(k >= _K)
        def _free_out_slot():
            out_copy(k - _K, slot).wait()

        @pl.when(fetch)
        def _fetch_global():
            g = pltpu.make_async_copy(gpe_hbm.at[row, col], gchunk, gsem)
            g.start()
            g.wait()

        obuf[slot] = ibuf[slot] + lbuf[...]

        @pl.when(fetch)
        def _add_global():
            obuf[slot] += gchunk[...] * tg

        out_copy(k, slot).start()

        @pl.when(k + _K - 1 < _N_SLICES)
        def _prefetch():
            in_copy(k + _K - 1, (k + _K - 1) % _K).start()

        return carry

    jax.lax.fori_loop(0, _N_SLICES, step, 0)

    for k in range(_N_SLICES - _K, _N_SLICES):
        out_copy(k, k % _K).wait()


def kernel(x, aspect_ratio, global_positional_embedding, local_positional_embedding, gate):
    bsz, n_tiles, num_tokens, embed_dim = x.shape
    ar = aspect_ratio.astype(jnp.int32)

    return pl.pallas_call(
        _body,
        in_specs=[
            pl.BlockSpec(memory_space=pltpu.SMEM),
            pl.BlockSpec(memory_space=pltpu.SMEM),
            pl.BlockSpec(memory_space=pltpu.MemorySpace.HBM),
            pl.BlockSpec(memory_space=pltpu.MemorySpace.HBM),
            pl.BlockSpec(memory_space=pltpu.MemorySpace.HBM),
        ],
        out_specs=pl.BlockSpec(memory_space=pltpu.MemorySpace.HBM),
        out_shape=jax.ShapeDtypeStruct((bsz, n_tiles, num_tokens, embed_dim), x.dtype),
        scratch_shapes=[
            pltpu.VMEM((_K, num_tokens, embed_dim), jnp.float32),
            pltpu.VMEM((_K, num_tokens, embed_dim), jnp.float32),
            pltpu.VMEM((num_tokens, embed_dim), jnp.float32),
            pltpu.VMEM((num_tokens, embed_dim), jnp.float32),
            pltpu.SemaphoreType.DMA((_K,)),
            pltpu.SemaphoreType.DMA((_K,)),
            pltpu.SemaphoreType.DMA,
            pltpu.SemaphoreType.DMA,
        ],
    )(ar, gate, x, local_positional_embedding, global_positional_embedding)
